# baseline (speedup 1.0000x reference)
import jax
import jax.numpy as jnp
from jax.experimental import pallas as pl
from jax.experimental.pallas import tpu as pltpu

_BM = 256
_NB = 4096 // _BM


def _fused_body(e_ref, tar_ref, src_ref, o_ref, t_ref):
    i = pl.program_id(0)

    @pl.when(i < _NB)
    def _stage1():
        acc = jnp.dot(tar_ref[...], e_ref[...],
                      preferred_element_type=jnp.float32,
                      precision=jax.lax.Precision.DEFAULT)
        t_ref[pl.ds(i * _BM, _BM), :] = acc.astype(jnp.bfloat16)

    @pl.when(i >= _NB)
    def _stage2():
        o_ref[...] = jnp.dot(src_ref[...], t_ref[...],
                             preferred_element_type=jnp.float32,
                             precision=jax.lax.Precision.DEFAULT)


def kernel(poi_embs, hg_poi_src, hg_poi_tar):
    n, k = hg_poi_src.shape
    _, d = poi_embs.shape
    return pl.pallas_call(
        _fused_body,
        grid=(2 * _NB,),
        in_specs=[
            pl.BlockSpec((k, d), lambda i: (0, 0)),
            pl.BlockSpec((_BM, k), lambda i: (jnp.minimum(i, _NB - 1), 0)),
            pl.BlockSpec((_BM, k), lambda i: (jnp.maximum(i - _NB, 0), 0)),
        ],
        out_specs=pl.BlockSpec((_BM, d), lambda i: (jnp.maximum(i - _NB, 0), 0)),
        out_shape=jax.ShapeDtypeStruct((n, d), jnp.float32),
        scratch_shapes=[pltpu.VMEM((k, d), jnp.bfloat16)],
        compiler_params=pltpu.CompilerParams(
            dimension_semantics=("arbitrary",),
        ),
    )(poi_embs, hg_poi_tar, hg_poi_src)


# speedup vs baseline: 1.0363x; 1.0363x over previous
import jax
import jax.numpy as jnp
from jax.experimental import pallas as pl
from jax.experimental.pallas import tpu as pltpu

_BM = 512
_NB = 4096 // _BM


def _fused_body(e_ref, tar_ref, src_ref, o_ref, t_ref):
    i = pl.program_id(0)

    @pl.when(i < _NB)
    def _stage1():
        acc = jnp.dot(tar_ref[...], e_ref[...],
                      preferred_element_type=jnp.float32,
                      precision=jax.lax.Precision.DEFAULT)
        t_ref[pl.ds(i * _BM, _BM), :] = acc.astype(jnp.bfloat16)

    @pl.when(i >= _NB)
    def _stage2():
        o_ref[...] = jnp.dot(src_ref[...], t_ref[...],
                             preferred_element_type=jnp.float32,
                             precision=jax.lax.Precision.DEFAULT)


def kernel(poi_embs, hg_poi_src, hg_poi_tar):
    n, k = hg_poi_src.shape
    _, d = poi_embs.shape
    return pl.pallas_call(
        _fused_body,
        grid=(2 * _NB,),
        in_specs=[
            pl.BlockSpec((k, d), lambda i: (0, 0)),
            pl.BlockSpec((_BM, k), lambda i: (jnp.minimum(i, _NB - 1), 0)),
            pl.BlockSpec((_BM, k), lambda i: (jnp.maximum(i - _NB, 0), 0)),
        ],
        out_specs=pl.BlockSpec((_BM, d), lambda i: (jnp.maximum(i - _NB, 0), 0)),
        out_shape=jax.ShapeDtypeStruct((n, d), jnp.float32),
        scratch_shapes=[pltpu.VMEM((k, d), jnp.bfloat16)],
        compiler_params=pltpu.CompilerParams(
            dimension_semantics=("arbitrary",),
            vmem_limit_bytes=66060288,
        ),
    )(poi_embs, hg_poi_tar, hg_poi_src)
